# baseline (speedup 1.0000x reference)
import functools

import jax
import jax.numpy as jnp
from jax import lax
from jax.experimental import pallas as pl
from jax.experimental.pallas import tpu as pltpu
from jax.experimental.pallas import tpu_sc as plsc

N_FIELDS = 26
VOCAB = 100000
DIM = 64
BATCH = 4096

NC = 2
NS = 16
NW = NC * NS
TOTAL = N_FIELDS * BATCH
CHUNK = 128
N_CHUNKS = TOTAL // CHUNK
CPW = N_CHUNKS // NW
CHUNKS_PER_FIELD = BATCH // CHUNK


@functools.partial(
    pl.kernel,
    mesh=plsc.VectorSubcoreMesh(core_axis_name="c", subcore_axis_name="s"),
    out_type=jax.ShapeDtypeStruct((TOTAL, DIM), jnp.float32),
    compiler_params=pltpu.CompilerParams(use_tc_tiling_on_sc=False),
    scratch_types=[
        pltpu.VMEM((CPW, CHUNK), jnp.int32),
        pltpu.VMEM((2, CHUNK, DIM), jnp.float32),
        pltpu.SemaphoreType.DMA,
        pltpu.SemaphoreType.DMA,
        pltpu.SemaphoreType.DMA,
        pltpu.SemaphoreType.DMA,
    ],
)
def _emb_lookup(xt_hbm, tbl_hbm, out_hbm, idx_v, rows_v, gs0, gs1, ws0, ws1):
    gsems = (gs0, gs1)
    wsems = (ws0, ws1)
    wid = lax.axis_index("s") * NC + lax.axis_index("c")
    chunk0 = wid * CPW

    pltpu.sync_copy(xt_hbm.at[wid], idx_v)

    for j in range(CPW):
        off = ((chunk0 + j) // CHUNKS_PER_FIELD) * VOCAB
        for k in range(CHUNK // 16):
            sl = pl.ds(k * 16, 16)
            idx_v[j, sl] = idx_v[j, sl] + off

    def start_gather(j):
        b = j & 1
        return pltpu.async_copy(tbl_hbm.at[idx_v.at[j]], rows_v.at[b], gsems[b])

    def start_write(j):
        b = j & 1
        dst = out_hbm.at[pl.ds((chunk0 + j) * CHUNK, CHUNK)]
        return pltpu.async_copy(rows_v.at[b], dst, wsems[b])

    g = start_gather(0)
    w_pend = [None, None]
    for j in range(CPW):
        b = j & 1
        g_next = None
        if j + 1 < CPW:
            if w_pend[1 - b] is not None:
                w_pend[1 - b].wait()
            g_next = start_gather(j + 1)
        g.wait()
        w_pend[b] = start_write(j)
        g = g_next
    w_pend[0].wait()
    w_pend[1].wait()


def kernel(X, tables):
    xt = X.astype(jnp.int32).T.reshape(NW, CPW, CHUNK)
    tbl = tables.reshape(N_FIELDS * VOCAB, DIM)
    out = _emb_lookup(xt, tbl)
    return out.reshape(N_FIELDS, BATCH, 1, DIM)


# speedup vs baseline: 2.9885x; 2.9885x over previous
import functools

import jax
import jax.numpy as jnp
from jax import lax
from jax.experimental import pallas as pl
from jax.experimental.pallas import tpu as pltpu
from jax.experimental.pallas import tpu_sc as plsc

N_FIELDS = 26
VOCAB = 100000
DIM = 64
BATCH = 4096

NW = 32
NBLK = N_FIELDS * 8
BPW = (NBLK + NW - 1) // NW
CHUNK_V = 4096
VMAIN = (VOCAB // 128) * 128
VTAIL = VOCAB - VMAIN
NCH = (VMAIN + CHUNK_V - 1) // CHUNK_V
NVREG = BATCH // 16


@functools.partial(
    pl.kernel,
    mesh=plsc.VectorSubcoreMesh(core_axis_name="c", subcore_axis_name="s"),
    out_type=jax.ShapeDtypeStruct((N_FIELDS, DIM, BATCH), jnp.float32),
    compiler_params=pltpu.CompilerParams(needs_layout_passes=False),
    scratch_types=[
        pltpu.VMEM((BATCH // 128, 128), jnp.int32),
        pltpu.VMEM((BATCH // 128, 128), jnp.int32),
        pltpu.VMEM((1, 32), jnp.int32),
        pltpu.VMEM((2, 8, CHUNK_V), jnp.float32),
        pltpu.VMEM((8, VTAIL), jnp.float32),
        pltpu.VMEM((8, BATCH), jnp.float32),
        pltpu.SemaphoreType.DMA,
        pltpu.SemaphoreType.DMA,
    ],
)
def _emb_sweep(sv_hbm, pm_hbm, phi_hbm, tbl_hbm, tail_hbm, out_hbm,
               sv_v, pm_v, phi_v, slab, tailb, outb, sem0, sem1):
    sems = (sem0, sem1)
    wid = lax.axis_index("s") * 2 + lax.axis_index("c")
    lane = lax.iota(jnp.int32, 16)

    def do_block(blk):
        i = blk // 8
        db8 = pl.multiple_of((blk % 8) * 8, 8)
        pltpu.sync_copy(sv_hbm.at[i], sv_v)
        pltpu.sync_copy(pm_hbm.at[i], pm_v)
        pltpu.sync_copy(phi_hbm.at[i], phi_v)
        pltpu.sync_copy(tail_hbm.at[i, pl.ds(db8, 8), :], tailb)
        phi_lo16 = phi_v[0, pl.ds(0, 16)]
        phi_hi16 = phi_v[0, pl.ds(16, 16)]

        def phi_at(c):
            vec = phi_lo16 if c < 16 else phi_hi16
            return jnp.sum(jnp.where(lane == (c % 16), vec, 0))

        def chunk_len(c):
            return min(CHUNK_V, VMAIN - c * CHUNK_V)

        def start_chunk(c):
            clen = chunk_len(c)
            src = tbl_hbm.at[i, pl.ds(db8, 8), pl.ds(c * CHUNK_V, clen)]
            if clen == CHUNK_V:
                dst = slab.at[c % 2]
            else:
                dst = slab.at[c % 2, :, pl.ds(0, clen)]
            return pltpu.async_copy(src, dst, sems[c % 2])

        def process_vreg(j, c, clen):
            row = j // 8
            col = (j % 8) * 16
            v = sv_v[row, pl.ds(col, 16)]
            b = pm_v[row, pl.ds(col, 16)]
            lv = v - c * CHUNK_V
            mask = (lv >= 0) & (lv < clen)
            lvc = jnp.where(mask, lv, 0)
            for dr in range(8):
                drv = jnp.full((16,), dr, jnp.int32)
                val = plsc.load_gather(slab.at[c % 2], [drv, lvc], mask=mask)
                plsc.store_scatter(outb, [drv, b], val, mask=mask)
            return jnp.int32(0)

        cp = start_chunk(0)
        for c in range(NCH):
            clen = chunk_len(c)
            if c + 1 < NCH:
                cp_next = start_chunk(c + 1)
            cp.wait()
            jlo = jnp.int32(0) if c == 0 else phi_at(c - 1) >> 4
            jhi = (phi_at(c) + 15) >> 4
            lax.fori_loop(jlo, jhi,
                          lambda j, carry: process_vreg(j, c, clen),
                          jnp.int32(0))
            if c + 1 < NCH:
                cp = cp_next

        def tail_body(j, carry):
            row = j // 8
            col = (j % 8) * 16
            v = sv_v[row, pl.ds(col, 16)]
            b = pm_v[row, pl.ds(col, 16)]
            lv = v - VMAIN
            mask = lv >= 0
            lvc = jnp.where(mask, lv, 0)
            for dr in range(8):
                drv = jnp.full((16,), dr, jnp.int32)
                val = plsc.load_gather(tailb, [drv, lvc], mask=mask)
                plsc.store_scatter(outb, [drv, b], val, mask=mask)
            return carry

        lax.fori_loop(phi_at(NCH - 1) >> 4, jnp.int32(NVREG), tail_body,
                      jnp.int32(0))
        pltpu.sync_copy(outb, out_hbm.at[i, pl.ds(db8, 8), :])

    def round_body(k, carry):
        blk = k * NW + wid

        @pl.when(blk < NBLK)
        def _():
            do_block(blk)

        return carry

    lax.fori_loop(0, BPW, round_body, jnp.int32(0))


def kernel(X, tables):
    xt = X.astype(jnp.int32).T
    perm = jnp.argsort(xt, axis=1).astype(jnp.int32)
    sv = jnp.take_along_axis(xt, perm, axis=1)
    sv3 = sv.reshape(N_FIELDS, BATCH // 128, 128)
    pm3 = perm.reshape(N_FIELDS, BATCH // 128, 128)
    edges = jnp.array(
        [min((c + 1) * CHUNK_V, VMAIN) for c in range(NCH)], jnp.int32)
    phi = jax.vmap(lambda row: jnp.searchsorted(row, edges))(sv)
    phi = jnp.pad(phi.astype(jnp.int32), ((0, 0), (0, 32 - NCH)))
    phi3 = phi.reshape(N_FIELDS, 1, 32)
    tbl_t = tables.transpose(0, 2, 1)
    tail_t = tbl_t[:, :, VMAIN:]
    out_t = _emb_sweep(sv3, pm3, phi3, tbl_t, tail_t)
    return out_t.transpose(0, 2, 1).reshape(N_FIELDS, BATCH, 1, DIM)
